# Initial kernel scaffold; baseline (speedup 1.0000x reference)
import jax
import jax.numpy as jnp
from jax.experimental import pallas as pl


def kernel(x, edge_index, W_l1, b_l1, W_r1, W_l2, b_l2, W_r2, W_cls, b_cls):
    raise NotImplementedError("write your pallas kernel here")


# speedup vs baseline: 4.8568x; 4.8568x over previous
import functools

import jax
import jax.numpy as jnp
from jax import lax
from jax.experimental import pallas as pl
from jax.experimental.pallas import tpu as pltpu
from jax.experimental.pallas import tpu_sc as plsc

N_NODES = 10000
N_PAD = 10240
D = 128
E = 320000
N_CLS = 40

NC, NS = 2, 16
NW = NC * NS
E_PER_TILE = E // NW
CH = 80
N_CHUNKS = E_PER_TILE // CH
ROWS_PER_TILE = N_PAD // NS
ZR = 32
BLK = 1024


def _sc_agg_body(hl_hbm, src_hbm, dst_hbm, aggp_hbm, degp_hbm,
                 agg_sh, deg_sh, sidx, didx, rows, zrow, ones, sem):
    cid = lax.axis_index("c")
    sid = lax.axis_index("s")

    zero16 = jnp.zeros((16,), jnp.float32)
    one16 = jnp.ones((16,), jnp.float32)
    for r in range(ZR):
        for j in range(D // 16):
            zrow[r, pl.ds(j * 16, 16)] = zero16
    for j in range(CH // 16):
        ones[pl.ds(j * 16, 16)] = one16

    r0 = sid * ROWS_PER_TILE
    for j in range(ROWS_PER_TILE // ZR):
        pltpu.sync_copy(zrow, agg_sh.at[pl.ds(r0 + j * ZR, ZR)])
    for j in range(ROWS_PER_TILE // D):
        pltpu.sync_copy(zrow.at[0], deg_sh.at[pl.ds(r0 + j * D, D)])
    plsc.subcore_barrier()

    ebase = (cid * NS + sid) * E_PER_TILE

    def chunk_body(c, _):
        off = ebase + c * CH
        pltpu.sync_copy(src_hbm.at[pl.ds(off, CH)], sidx)
        pltpu.sync_copy(dst_hbm.at[pl.ds(off, CH)], didx)
        pltpu.async_copy(hl_hbm.at[sidx], rows, sem).wait()
        pltpu.sync_copy(rows, agg_sh.at[didx], add=True)
        pltpu.sync_copy(ones, deg_sh.at[didx], add=True)
        return 0

    lax.fori_loop(0, N_CHUNKS, chunk_body, 0)
    plsc.subcore_barrier()

    pltpu.sync_copy(agg_sh.at[pl.ds(r0, ROWS_PER_TILE)],
                    aggp_hbm.at[cid, pl.ds(r0, ROWS_PER_TILE)])
    pltpu.sync_copy(deg_sh.at[pl.ds(r0, ROWS_PER_TILE)],
                    degp_hbm.at[cid, pl.ds(r0, ROWS_PER_TILE)])


@functools.lru_cache(maxsize=None)
def _sc_agg():
    return pl.kernel(
        _sc_agg_body,
        out_type=(jax.ShapeDtypeStruct((NC, N_PAD, D), jnp.float32),
                  jax.ShapeDtypeStruct((NC, N_PAD), jnp.float32)),
        mesh=plsc.VectorSubcoreMesh(core_axis_name="c", subcore_axis_name="s"),
        scratch_types=(
            pltpu.VMEM_SHARED((N_PAD, D), jnp.float32),
            pltpu.VMEM_SHARED((N_PAD,), jnp.float32),
            pltpu.VMEM((CH,), jnp.int32),
            pltpu.VMEM((CH,), jnp.int32),
            pltpu.VMEM((CH, D), jnp.float32),
            pltpu.VMEM((ZR, D), jnp.float32),
            pltpu.VMEM((CH,), jnp.float32),
            pltpu.SemaphoreType.DMA,
        ),
    )


def _lin_body(h_ref, wl_ref, bl_ref, wr_ref, hl_ref, hr_ref):
    h = h_ref[...]
    hl_ref[...] = jnp.dot(h, wl_ref[...], preferred_element_type=jnp.float32) + bl_ref[...]
    hr_ref[...] = jnp.dot(h, wr_ref[...], preferred_element_type=jnp.float32)


def _lin(h, wl, bl, wr):
    return pl.pallas_call(
        _lin_body,
        grid=(N_PAD // BLK,),
        in_specs=[pl.BlockSpec((BLK, D), lambda i: (i, 0)),
                  pl.BlockSpec((D, D), lambda i: (0, 0)),
                  pl.BlockSpec((1, D), lambda i: (0, 0)),
                  pl.BlockSpec((D, D), lambda i: (0, 0))],
        out_specs=[pl.BlockSpec((BLK, D), lambda i: (i, 0)),
                   pl.BlockSpec((BLK, D), lambda i: (i, 0))],
        out_shape=[jax.ShapeDtypeStruct((N_PAD, D), jnp.float32),
                   jax.ShapeDtypeStruct((N_PAD, D), jnp.float32)],
    )(h, wl, bl, wr)


def _mid_body(aggp_ref, degp_ref, hr_ref, wl_ref, bl_ref, wr_ref,
              hl2_ref, hr2_ref, inv_ref):
    p = aggp_ref[0] + aggp_ref[1]
    dsum = degp_ref[0] + degp_ref[1]
    inv = jnp.where(dsum > 0, 1.0 / jnp.maximum(dsum, 1.0), 0.0)
    h2 = jnp.maximum(p * inv + hr_ref[...], 0.0)
    hl2_ref[...] = jnp.dot(h2, wl_ref[...], preferred_element_type=jnp.float32) + bl_ref[...]
    hr2_ref[...] = jnp.dot(h2, wr_ref[...], preferred_element_type=jnp.float32)
    inv_ref[...] = inv


def _mid(aggp, degp, hr1, wl, bl, wr):
    return pl.pallas_call(
        _mid_body,
        grid=(N_PAD // BLK,),
        in_specs=[pl.BlockSpec((NC, BLK, D), lambda i: (0, i, 0)),
                  pl.BlockSpec((NC, BLK, 1), lambda i: (0, i, 0)),
                  pl.BlockSpec((BLK, D), lambda i: (i, 0)),
                  pl.BlockSpec((D, D), lambda i: (0, 0)),
                  pl.BlockSpec((1, D), lambda i: (0, 0)),
                  pl.BlockSpec((D, D), lambda i: (0, 0))],
        out_specs=[pl.BlockSpec((BLK, D), lambda i: (i, 0)),
                   pl.BlockSpec((BLK, D), lambda i: (i, 0)),
                   pl.BlockSpec((BLK, 1), lambda i: (i, 0))],
        out_shape=[jax.ShapeDtypeStruct((N_PAD, D), jnp.float32),
                   jax.ShapeDtypeStruct((N_PAD, D), jnp.float32),
                   jax.ShapeDtypeStruct((N_PAD, 1), jnp.float32)],
    )(aggp, degp, hr1, wl, bl, wr)


def _fin_body(aggp_ref, inv_ref, hr_ref, wc_ref, bc_ref, out_ref):
    p = aggp_ref[0] + aggp_ref[1]
    h3 = jnp.maximum(p * inv_ref[...] + hr_ref[...], 0.0)
    out_ref[...] = jnp.dot(h3, wc_ref[...], preferred_element_type=jnp.float32) + bc_ref[...]


def _fin(aggp, inv, hr2, wc, bc):
    return pl.pallas_call(
        _fin_body,
        grid=(N_PAD // BLK,),
        in_specs=[pl.BlockSpec((NC, BLK, D), lambda i: (0, i, 0)),
                  pl.BlockSpec((BLK, 1), lambda i: (i, 0)),
                  pl.BlockSpec((BLK, D), lambda i: (i, 0)),
                  pl.BlockSpec((D, N_CLS), lambda i: (0, 0)),
                  pl.BlockSpec((1, N_CLS), lambda i: (0, 0))],
        out_specs=pl.BlockSpec((BLK, N_CLS), lambda i: (i, 0)),
        out_shape=jax.ShapeDtypeStruct((N_PAD, N_CLS), jnp.float32),
    )(aggp, inv, hr2, wc, bc)


def kernel(x, edge_index, W_l1, b_l1, W_r1, W_l2, b_l2, W_r2, W_cls, b_cls):
    n = x.shape[0]
    src = edge_index[0].astype(jnp.int32)
    dst = edge_index[1].astype(jnp.int32)
    xp = jnp.zeros((N_PAD, D), jnp.float32).at[:n].set(x)

    hl1, hr1 = _lin(xp, W_l1, b_l1.reshape(1, D), W_r1)
    aggp1, degp1 = _sc_agg()(hl1, src, dst)
    hl2, hr2, inv = _mid(aggp1, degp1.reshape(NC, N_PAD, 1), hr1,
                         W_l2, b_l2.reshape(1, D), W_r2)
    aggp2, _ = _sc_agg()(hl2, src, dst)
    out = _fin(aggp2, inv, hr2, W_cls, b_cls.reshape(1, N_CLS))
    return out[:n]


# speedup vs baseline: 10.6650x; 2.1959x over previous
import functools

import jax
import jax.numpy as jnp
from jax import lax
from jax.experimental import pallas as pl
from jax.experimental.pallas import tpu as pltpu
from jax.experimental.pallas import tpu_sc as plsc

N_NODES = 10000
N_PAD = 10240
D = 128
E = 320000
N_CLS = 40

NC, NS = 2, 16
NW = NC * NS
E_PER_TILE = E // NW
CH = 80
N_CHUNKS = E_PER_TILE // CH
N_PAIRS = (N_CHUNKS - 1) // 2
ROWS_PER_TILE = N_PAD // NS
BLK = 1024


def _sc_agg_body(with_deg, hl_hbm, src_hbm, dst_hbm, aggp_hbm, degp_hbm,
                 agg_sh, deg_sh, sall, dall, rows0, rows1, ones,
                 sem0, sem1):
    cid = lax.axis_index("c")
    sid = lax.axis_index("s")
    wid = cid * NS + sid

    pltpu.sync_copy(src_hbm.at[wid], sall)
    pltpu.sync_copy(dst_hbm.at[wid], dall)

    zero16 = jnp.zeros((16,), jnp.float32)
    for r in range(CH):
        for j in range(D // 16):
            rows0[r, pl.ds(j * 16, 16)] = zero16
    if with_deg:
        one16 = jnp.ones((16,), jnp.float32)
        for j in range(CH // 16):
            ones[pl.ds(j * 16, 16)] = one16
        ones[pl.ds(CH - 16, 16)] = one16

    r0 = sid * ROWS_PER_TILE
    for j in range(ROWS_PER_TILE // CH):
        pltpu.sync_copy(rows0, agg_sh.at[pl.ds(r0 + j * CH, CH)])
    if with_deg:
        for j in range(ROWS_PER_TILE // D):
            pltpu.sync_copy(rows0.at[0], deg_sh.at[pl.ds(r0 + j * D, D)])
    plsc.subcore_barrier()

    def start(c, rows, sem):
        pltpu.async_copy(hl_hbm.at[sall.at[pl.ds(c * CH, CH)]], rows, sem)

    def finish(c, rows, sem):
        pltpu.make_async_copy(hl_hbm.at[sall.at[pl.ds(c * CH, CH)]],
                              rows, sem).wait()
        pltpu.sync_copy(rows, agg_sh.at[dall.at[c]], add=True)
        if with_deg:
            pltpu.sync_copy(ones, deg_sh.at[dall.at[c]], add=True)

    start(0, rows0, sem0)

    def pair_body(i, carry):
        c = 2 * i
        start(c + 1, rows1, sem1)
        finish(c, rows0, sem0)
        start(c + 2, rows0, sem0)
        finish(c + 1, rows1, sem1)
        return carry

    lax.fori_loop(0, N_PAIRS, pair_body, 0)
    finish(N_CHUNKS - 1, rows0, sem0)
    plsc.subcore_barrier()

    pltpu.sync_copy(agg_sh.at[pl.ds(r0, ROWS_PER_TILE)],
                    aggp_hbm.at[cid, pl.ds(r0, ROWS_PER_TILE)])
    if with_deg:
        pltpu.sync_copy(deg_sh.at[pl.ds(r0, ROWS_PER_TILE)],
                        degp_hbm.at[cid, pl.ds(r0, ROWS_PER_TILE)])


@functools.lru_cache(maxsize=None)
def _sc_agg(with_deg):
    return pl.kernel(
        functools.partial(_sc_agg_body, with_deg),
        out_type=(jax.ShapeDtypeStruct((NC, N_PAD, D), jnp.float32),
                  jax.ShapeDtypeStruct((NC, N_PAD), jnp.float32)),
        mesh=plsc.VectorSubcoreMesh(core_axis_name="c", subcore_axis_name="s"),
        scratch_types=(
            pltpu.VMEM_SHARED((N_PAD, D), jnp.float32),
            pltpu.VMEM_SHARED((N_PAD,), jnp.float32),
            pltpu.VMEM((E_PER_TILE,), jnp.int32),
            pltpu.VMEM((N_CHUNKS, CH), jnp.int32),
            pltpu.VMEM((CH, D), jnp.float32),
            pltpu.VMEM((CH, D), jnp.float32),
            pltpu.VMEM((CH,), jnp.float32),
            pltpu.SemaphoreType.DMA,
            pltpu.SemaphoreType.DMA,
        ),
    )


def _lin_body(h_ref, wl_ref, bl_ref, wr_ref, hl_ref, hr_ref):
    h = h_ref[...]
    hl_ref[...] = jnp.dot(h, wl_ref[...], preferred_element_type=jnp.float32) + bl_ref[...]
    hr_ref[...] = jnp.dot(h, wr_ref[...], preferred_element_type=jnp.float32)


def _lin(h, wl, bl, wr):
    return pl.pallas_call(
        _lin_body,
        grid=(N_PAD // BLK,),
        in_specs=[pl.BlockSpec((BLK, D), lambda i: (i, 0)),
                  pl.BlockSpec((D, D), lambda i: (0, 0)),
                  pl.BlockSpec((1, D), lambda i: (0, 0)),
                  pl.BlockSpec((D, D), lambda i: (0, 0))],
        out_specs=[pl.BlockSpec((BLK, D), lambda i: (i, 0)),
                   pl.BlockSpec((BLK, D), lambda i: (i, 0))],
        out_shape=[jax.ShapeDtypeStruct((N_PAD, D), jnp.float32),
                   jax.ShapeDtypeStruct((N_PAD, D), jnp.float32)],
    )(h, wl, bl, wr)


def _mid_body(aggp_ref, degp_ref, hr_ref, wl_ref, bl_ref, wr_ref,
              hl2_ref, hr2_ref, inv_ref):
    p = aggp_ref[0] + aggp_ref[1]
    dsum = degp_ref[0] + degp_ref[1]
    inv = jnp.where(dsum > 0, 1.0 / jnp.maximum(dsum, 1.0), 0.0)
    h2 = jnp.maximum(p * inv + hr_ref[...], 0.0)
    hl2_ref[...] = jnp.dot(h2, wl_ref[...], preferred_element_type=jnp.float32) + bl_ref[...]
    hr2_ref[...] = jnp.dot(h2, wr_ref[...], preferred_element_type=jnp.float32)
    inv_ref[...] = inv


def _mid(aggp, degp, hr1, wl, bl, wr):
    return pl.pallas_call(
        _mid_body,
        grid=(N_PAD // BLK,),
        in_specs=[pl.BlockSpec((NC, BLK, D), lambda i: (0, i, 0)),
                  pl.BlockSpec((NC, BLK, 1), lambda i: (0, i, 0)),
                  pl.BlockSpec((BLK, D), lambda i: (i, 0)),
                  pl.BlockSpec((D, D), lambda i: (0, 0)),
                  pl.BlockSpec((1, D), lambda i: (0, 0)),
                  pl.BlockSpec((D, D), lambda i: (0, 0))],
        out_specs=[pl.BlockSpec((BLK, D), lambda i: (i, 0)),
                   pl.BlockSpec((BLK, D), lambda i: (i, 0)),
                   pl.BlockSpec((BLK, 1), lambda i: (i, 0))],
        out_shape=[jax.ShapeDtypeStruct((N_PAD, D), jnp.float32),
                   jax.ShapeDtypeStruct((N_PAD, D), jnp.float32),
                   jax.ShapeDtypeStruct((N_PAD, 1), jnp.float32)],
    )(aggp, degp, hr1, wl, bl, wr)


def _fin_body(aggp_ref, inv_ref, hr_ref, wc_ref, bc_ref, out_ref):
    p = aggp_ref[0] + aggp_ref[1]
    h3 = jnp.maximum(p * inv_ref[...] + hr_ref[...], 0.0)
    out_ref[...] = jnp.dot(h3, wc_ref[...], preferred_element_type=jnp.float32) + bc_ref[...]


def _fin(aggp, inv, hr2, wc, bc):
    return pl.pallas_call(
        _fin_body,
        grid=(N_PAD // BLK,),
        in_specs=[pl.BlockSpec((NC, BLK, D), lambda i: (0, i, 0)),
                  pl.BlockSpec((BLK, 1), lambda i: (i, 0)),
                  pl.BlockSpec((BLK, D), lambda i: (i, 0)),
                  pl.BlockSpec((D, N_CLS), lambda i: (0, 0)),
                  pl.BlockSpec((1, N_CLS), lambda i: (0, 0))],
        out_specs=pl.BlockSpec((BLK, N_CLS), lambda i: (i, 0)),
        out_shape=jax.ShapeDtypeStruct((N_PAD, N_CLS), jnp.float32),
    )(aggp, inv, hr2, wc, bc)


def kernel(x, edge_index, W_l1, b_l1, W_r1, W_l2, b_l2, W_r2, W_cls, b_cls):
    n = x.shape[0]
    src = edge_index[0].astype(jnp.int32).reshape(NW, E_PER_TILE)
    dst = edge_index[1].astype(jnp.int32).reshape(NW, N_CHUNKS, CH)
    xp = jnp.zeros((N_PAD, D), jnp.float32).at[:n].set(x)

    hl1, hr1 = _lin(xp, W_l1, b_l1.reshape(1, D), W_r1)
    aggp1, degp1 = _sc_agg(True)(hl1, src, dst)
    hl2, hr2, inv = _mid(aggp1, degp1.reshape(NC, N_PAD, 1), hr1,
                         W_l2, b_l2.reshape(1, D), W_r2)
    aggp2, _ = _sc_agg(False)(hl2, src, dst)
    out = _fin(aggp2, inv, hr2, W_cls, b_cls.reshape(1, N_CLS))
    return out[:n]


# speedup vs baseline: 10.7755x; 1.0104x over previous
import functools

import jax
import jax.numpy as jnp
from jax import lax
from jax.experimental import pallas as pl
from jax.experimental.pallas import tpu as pltpu
from jax.experimental.pallas import tpu_sc as plsc

N_NODES = 10000
N_PAD = 10240
D = 128
E = 320000
N_CLS = 40

NC, NS = 2, 16
NW = NC * NS
E_PER_TILE = E // NW
CH = 80
N_CHUNKS = E_PER_TILE // CH
N_PAIRS = (N_CHUNKS - 1) // 2
ROWS_PER_TILE = N_PAD // NS
BLK = 1024


def _sc_agg_body(with_deg, hl_hbm, src_hbm, dst_hbm, aggp_hbm, degp_hbm,
                 agg_sh, deg_sh, sall, dall, rows0, rows1, ones,
                 sem0, sem1, dsem):
    cid = lax.axis_index("c")
    sid = lax.axis_index("s")
    wid = cid * NS + sid

    pltpu.sync_copy(src_hbm.at[wid], sall)
    pltpu.sync_copy(dst_hbm.at[wid], dall)

    zero16 = jnp.zeros((16,), jnp.float32)
    for r in range(CH):
        for j in range(D // 16):
            rows0[r, pl.ds(j * 16, 16)] = zero16
    if with_deg:
        one16 = jnp.ones((16,), jnp.float32)
        for j in range(CH // 16):
            ones[pl.ds(j * 16, 16)] = one16
        ones[pl.ds(CH - 16, 16)] = one16

    r0 = sid * ROWS_PER_TILE
    for j in range(ROWS_PER_TILE // CH):
        pltpu.sync_copy(rows0, agg_sh.at[pl.ds(r0 + j * CH, CH)])
    if with_deg:
        for j in range(ROWS_PER_TILE // D):
            pltpu.sync_copy(rows0.at[0], deg_sh.at[pl.ds(r0 + j * D, D)])
    plsc.subcore_barrier()

    def start(c, rows, sem):
        pltpu.async_copy(hl_hbm.at[sall.at[pl.ds(c * CH, CH)]], rows, sem)

    def finish(c, rows, sem):
        pltpu.make_async_copy(hl_hbm.at[sall.at[pl.ds(c * CH, CH)]],
                              rows, sem).wait()
        pltpu.sync_copy(rows, agg_sh.at[dall.at[c]], add=True)
        if with_deg:
            pltpu.async_copy(ones, deg_sh.at[dall.at[c]], dsem, add=True)

    start(0, rows0, sem0)

    def pair_body(i, carry):
        c = 2 * i
        start(c + 1, rows1, sem1)
        finish(c, rows0, sem0)
        start(c + 2, rows0, sem0)
        finish(c + 1, rows1, sem1)
        return carry

    lax.fori_loop(0, N_PAIRS, pair_body, 0)
    finish(N_CHUNKS - 1, rows0, sem0)
    if with_deg:
        def drain_body(c, carry):
            pltpu.make_async_copy(ones, deg_sh.at[dall.at[c]], dsem).wait()
            return carry

        lax.fori_loop(0, N_CHUNKS, drain_body, 0)
    plsc.subcore_barrier()

    pltpu.sync_copy(agg_sh.at[pl.ds(r0, ROWS_PER_TILE)],
                    aggp_hbm.at[cid, pl.ds(r0, ROWS_PER_TILE)])
    if with_deg:
        pltpu.sync_copy(deg_sh.at[pl.ds(r0, ROWS_PER_TILE)],
                        degp_hbm.at[cid, pl.ds(r0, ROWS_PER_TILE)])


@functools.lru_cache(maxsize=None)
def _sc_agg(with_deg):
    return pl.kernel(
        functools.partial(_sc_agg_body, with_deg),
        out_type=(jax.ShapeDtypeStruct((NC, N_PAD, D), jnp.float32),
                  jax.ShapeDtypeStruct((NC, N_PAD), jnp.float32)),
        mesh=plsc.VectorSubcoreMesh(core_axis_name="c", subcore_axis_name="s"),
        scratch_types=(
            pltpu.VMEM_SHARED((N_PAD, D), jnp.float32),
            pltpu.VMEM_SHARED((N_PAD,), jnp.float32),
            pltpu.VMEM((E_PER_TILE,), jnp.int32),
            pltpu.VMEM((N_CHUNKS, CH), jnp.int32),
            pltpu.VMEM((CH, D), jnp.float32),
            pltpu.VMEM((CH, D), jnp.float32),
            pltpu.VMEM((CH,), jnp.float32),
            pltpu.SemaphoreType.DMA,
            pltpu.SemaphoreType.DMA,
            pltpu.SemaphoreType.DMA,
        ),
    )


def _lin_body(h_ref, wl_ref, bl_ref, wr_ref, hl_ref, hr_ref):
    h = h_ref[...]
    hl_ref[...] = jnp.dot(h, wl_ref[...], preferred_element_type=jnp.float32) + bl_ref[...]
    hr_ref[...] = jnp.dot(h, wr_ref[...], preferred_element_type=jnp.float32)


def _lin(h, wl, bl, wr):
    return pl.pallas_call(
        _lin_body,
        grid=(N_PAD // BLK,),
        in_specs=[pl.BlockSpec((BLK, D), lambda i: (i, 0)),
                  pl.BlockSpec((D, D), lambda i: (0, 0)),
                  pl.BlockSpec((1, D), lambda i: (0, 0)),
                  pl.BlockSpec((D, D), lambda i: (0, 0))],
        out_specs=[pl.BlockSpec((BLK, D), lambda i: (i, 0)),
                   pl.BlockSpec((BLK, D), lambda i: (i, 0))],
        out_shape=[jax.ShapeDtypeStruct((N_PAD, D), jnp.float32),
                   jax.ShapeDtypeStruct((N_PAD, D), jnp.float32)],
    )(h, wl, bl, wr)


def _mid_body(aggp_ref, degp_ref, hr_ref, wl_ref, bl_ref, wr_ref,
              hl2_ref, hr2_ref, inv_ref):
    p = aggp_ref[0] + aggp_ref[1]
    dsum = degp_ref[0] + degp_ref[1]
    inv = jnp.where(dsum > 0, 1.0 / jnp.maximum(dsum, 1.0), 0.0)
    h2 = jnp.maximum(p * inv + hr_ref[...], 0.0)
    hl2_ref[...] = jnp.dot(h2, wl_ref[...], preferred_element_type=jnp.float32) + bl_ref[...]
    hr2_ref[...] = jnp.dot(h2, wr_ref[...], preferred_element_type=jnp.float32)
    inv_ref[...] = inv


def _mid(aggp, degp, hr1, wl, bl, wr):
    return pl.pallas_call(
        _mid_body,
        grid=(N_PAD // BLK,),
        in_specs=[pl.BlockSpec((NC, BLK, D), lambda i: (0, i, 0)),
                  pl.BlockSpec((NC, BLK, 1), lambda i: (0, i, 0)),
                  pl.BlockSpec((BLK, D), lambda i: (i, 0)),
                  pl.BlockSpec((D, D), lambda i: (0, 0)),
                  pl.BlockSpec((1, D), lambda i: (0, 0)),
                  pl.BlockSpec((D, D), lambda i: (0, 0))],
        out_specs=[pl.BlockSpec((BLK, D), lambda i: (i, 0)),
                   pl.BlockSpec((BLK, D), lambda i: (i, 0)),
                   pl.BlockSpec((BLK, 1), lambda i: (i, 0))],
        out_shape=[jax.ShapeDtypeStruct((N_PAD, D), jnp.float32),
                   jax.ShapeDtypeStruct((N_PAD, D), jnp.float32),
                   jax.ShapeDtypeStruct((N_PAD, 1), jnp.float32)],
    )(aggp, degp, hr1, wl, bl, wr)


def _fin_body(aggp_ref, inv_ref, hr_ref, wc_ref, bc_ref, out_ref):
    p = aggp_ref[0] + aggp_ref[1]
    h3 = jnp.maximum(p * inv_ref[...] + hr_ref[...], 0.0)
    out_ref[...] = jnp.dot(h3, wc_ref[...], preferred_element_type=jnp.float32) + bc_ref[...]


def _fin(aggp, inv, hr2, wc, bc):
    return pl.pallas_call(
        _fin_body,
        grid=(N_PAD // BLK,),
        in_specs=[pl.BlockSpec((NC, BLK, D), lambda i: (0, i, 0)),
                  pl.BlockSpec((BLK, 1), lambda i: (i, 0)),
                  pl.BlockSpec((BLK, D), lambda i: (i, 0)),
                  pl.BlockSpec((D, N_CLS), lambda i: (0, 0)),
                  pl.BlockSpec((1, N_CLS), lambda i: (0, 0))],
        out_specs=pl.BlockSpec((BLK, N_CLS), lambda i: (i, 0)),
        out_shape=jax.ShapeDtypeStruct((N_PAD, N_CLS), jnp.float32),
    )(aggp, inv, hr2, wc, bc)


def kernel(x, edge_index, W_l1, b_l1, W_r1, W_l2, b_l2, W_r2, W_cls, b_cls):
    n = x.shape[0]
    src = edge_index[0].astype(jnp.int32).reshape(NW, E_PER_TILE)
    dst = edge_index[1].astype(jnp.int32).reshape(NW, N_CHUNKS, CH)
    xp = jnp.zeros((N_PAD, D), jnp.float32).at[:n].set(x)

    hl1, hr1 = _lin(xp, W_l1, b_l1.reshape(1, D), W_r1)
    aggp1, degp1 = _sc_agg(True)(hl1, src, dst)
    degp3 = degp1.reshape(NC, N_PAD, 1)
    hl2, hr2, inv = _mid(aggp1, degp3, hr1, W_l2, b_l2.reshape(1, D), W_r2)
    aggp2, _ = _sc_agg(False)(hl2, src, dst)
    out = _fin(aggp2, inv, hr2, W_cls, b_cls.reshape(1, N_CLS))
    return out[:n]


# speedup vs baseline: 11.2092x; 1.0402x over previous
import functools

import jax
import jax.numpy as jnp
from jax import lax
from jax.experimental import pallas as pl
from jax.experimental.pallas import tpu as pltpu
from jax.experimental.pallas import tpu_sc as plsc

N_NODES = 10000
N_PAD = 10240
D = 128
E = 320000
N_CLS = 40

NC, NS = 2, 16
NW = NC * NS
E_PER_TILE = E // NW
CH = 80
N_CHUNKS = E_PER_TILE // CH
N_PAIRS = (N_CHUNKS - 1) // 2
ROWS_PER_TILE = N_PAD // NS
BLK = 1024


def _sc_agg_body(with_deg, hl_hbm, src_hbm, dst_hbm, aggp_hbm, degp_hbm,
                 agg_sh, deg_sh, sall, dall, rows0, rows1, ones,
                 sem0, sem1, dsem):
    cid = lax.axis_index("c")
    sid = lax.axis_index("s")
    wid = cid * NS + sid

    pltpu.sync_copy(src_hbm.at[wid], sall)
    pltpu.sync_copy(dst_hbm.at[wid], dall)

    zero16 = jnp.zeros((16,), jnp.float32)
    for r in range(CH):
        for j in range(D // 16):
            rows0[r, pl.ds(j * 16, 16)] = zero16
    if with_deg:
        one16 = jnp.ones((16,), jnp.float32)
        for j in range(CH // 16):
            ones[pl.ds(j * 16, 16)] = one16
        ones[pl.ds(CH - 16, 16)] = one16

    r0 = sid * ROWS_PER_TILE
    for j in range(ROWS_PER_TILE // CH):
        pltpu.sync_copy(rows0, agg_sh.at[pl.ds(r0 + j * CH, CH)])
    if with_deg:
        for j in range(ROWS_PER_TILE // D):
            pltpu.sync_copy(rows0.at[0], deg_sh.at[pl.ds(r0 + j * D, D)])
    plsc.subcore_barrier()

    def start(c, rows, sem):
        pltpu.async_copy(hl_hbm.at[sall.at[pl.ds(c * CH, CH)]], rows, sem)

    def finish(c, rows, sem):
        pltpu.make_async_copy(hl_hbm.at[sall.at[pl.ds(c * CH, CH)]],
                              rows, sem).wait()
        pltpu.sync_copy(rows, agg_sh.at[dall.at[c]], add=True)
        if with_deg:
            pltpu.async_copy(ones, deg_sh.at[dall.at[c]], dsem, add=True)

    start(0, rows0, sem0)

    def pair_body(i, carry):
        c = 2 * i
        start(c + 1, rows1, sem1)
        finish(c, rows0, sem0)
        start(c + 2, rows0, sem0)
        finish(c + 1, rows1, sem1)
        return carry

    lax.fori_loop(0, N_PAIRS, pair_body, 0)
    finish(N_CHUNKS - 1, rows0, sem0)
    if with_deg:
        def drain_body(c, carry):
            pltpu.make_async_copy(ones, deg_sh.at[dall.at[c]], dsem).wait()
            return carry

        lax.fori_loop(0, N_CHUNKS, drain_body, 0)
    plsc.subcore_barrier()

    pltpu.sync_copy(agg_sh.at[pl.ds(r0, ROWS_PER_TILE)],
                    aggp_hbm.at[cid, pl.ds(r0, ROWS_PER_TILE)])
    if with_deg:
        pltpu.sync_copy(deg_sh.at[pl.ds(r0, ROWS_PER_TILE)],
                        degp_hbm.at[cid, pl.ds(r0, ROWS_PER_TILE)])


@functools.lru_cache(maxsize=None)
def _sc_agg(with_deg):
    return pl.kernel(
        functools.partial(_sc_agg_body, with_deg),
        out_type=(jax.ShapeDtypeStruct((NC, N_PAD, D), jnp.float32),
                  jax.ShapeDtypeStruct((NC, N_PAD), jnp.float32)),
        mesh=plsc.VectorSubcoreMesh(core_axis_name="c", subcore_axis_name="s"),
        scratch_types=(
            pltpu.VMEM_SHARED((N_PAD, D), jnp.float32),
            pltpu.VMEM_SHARED((N_PAD,), jnp.float32),
            pltpu.VMEM((E_PER_TILE,), jnp.int32),
            pltpu.VMEM((N_CHUNKS, CH), jnp.int32),
            pltpu.VMEM((CH, D), jnp.float32),
            pltpu.VMEM((CH, D), jnp.float32),
            pltpu.VMEM((CH,), jnp.float32),
            pltpu.SemaphoreType.DMA,
            pltpu.SemaphoreType.DMA,
            pltpu.SemaphoreType.DMA,
        ),
    )


def _mid_body(aggp_ref, degp_ref, x_ref, wl_ref, bl_ref, wr_ref,
              h2_ref, inv_ref):
    p = aggp_ref[0] + aggp_ref[1]
    dsum = degp_ref[0] + degp_ref[1]
    inv = jnp.where(dsum > 0, 1.0 / jnp.maximum(dsum, 1.0), 0.0)
    mask = jnp.where(dsum > 0, 1.0, 0.0)
    agg = jnp.dot(p * inv, wl_ref[...], preferred_element_type=jnp.float32) + bl_ref[...] * mask
    hr = jnp.dot(x_ref[...], wr_ref[...], preferred_element_type=jnp.float32)
    h2_ref[...] = jnp.maximum(agg + hr, 0.0)
    inv_ref[...] = inv


def _mid(aggp, degp, xp, wl, bl, wr):
    return pl.pallas_call(
        _mid_body,
        grid=(N_PAD // BLK,),
        in_specs=[pl.BlockSpec((NC, BLK, D), lambda i: (0, i, 0)),
                  pl.BlockSpec((NC, BLK, 1), lambda i: (0, i, 0)),
                  pl.BlockSpec((BLK, D), lambda i: (i, 0)),
                  pl.BlockSpec((D, D), lambda i: (0, 0)),
                  pl.BlockSpec((1, D), lambda i: (0, 0)),
                  pl.BlockSpec((D, D), lambda i: (0, 0))],
        out_specs=[pl.BlockSpec((BLK, D), lambda i: (i, 0)),
                   pl.BlockSpec((BLK, 1), lambda i: (i, 0))],
        out_shape=[jax.ShapeDtypeStruct((N_PAD, D), jnp.float32),
                   jax.ShapeDtypeStruct((N_PAD, 1), jnp.float32)],
    )(aggp, degp, xp, wl, bl, wr)


def _fin_body(aggp_ref, inv_ref, h2_ref, wl_ref, bl_ref, wr_ref, wc_ref,
              bc_ref, out_ref):
    p = aggp_ref[0] + aggp_ref[1]
    inv = inv_ref[...]
    mask = jnp.where(inv > 0, 1.0, 0.0)
    agg = jnp.dot(p * inv, wl_ref[...], preferred_element_type=jnp.float32) + bl_ref[...] * mask
    hr = jnp.dot(h2_ref[...], wr_ref[...], preferred_element_type=jnp.float32)
    h3 = jnp.maximum(agg + hr, 0.0)
    out_ref[...] = jnp.dot(h3, wc_ref[...], preferred_element_type=jnp.float32) + bc_ref[...]


def _fin(aggp, inv, h2, wl, bl, wr, wc, bc):
    return pl.pallas_call(
        _fin_body,
        grid=(N_PAD // BLK,),
        in_specs=[pl.BlockSpec((NC, BLK, D), lambda i: (0, i, 0)),
                  pl.BlockSpec((BLK, 1), lambda i: (i, 0)),
                  pl.BlockSpec((BLK, D), lambda i: (i, 0)),
                  pl.BlockSpec((D, D), lambda i: (0, 0)),
                  pl.BlockSpec((1, D), lambda i: (0, 0)),
                  pl.BlockSpec((D, D), lambda i: (0, 0)),
                  pl.BlockSpec((D, N_CLS), lambda i: (0, 0)),
                  pl.BlockSpec((1, N_CLS), lambda i: (0, 0))],
        out_specs=pl.BlockSpec((BLK, N_CLS), lambda i: (i, 0)),
        out_shape=jax.ShapeDtypeStruct((N_PAD, N_CLS), jnp.float32),
    )(aggp, inv, h2, wl, bl, wr, wc, bc)


def kernel(x, edge_index, W_l1, b_l1, W_r1, W_l2, b_l2, W_r2, W_cls, b_cls):
    n = x.shape[0]
    src = edge_index[0].astype(jnp.int32).reshape(NW, E_PER_TILE)
    dst = edge_index[1].astype(jnp.int32).reshape(NW, N_CHUNKS, CH)
    xp = jnp.zeros((N_PAD, D), jnp.float32).at[:n].set(x)

    aggp1, degp1 = _sc_agg(True)(xp, src, dst)
    degp3 = degp1.reshape(NC, N_PAD, 1)
    h2, inv = _mid(aggp1, degp3, xp, W_l1, b_l1.reshape(1, D), W_r1)
    aggp2, _ = _sc_agg(False)(h2, src, dst)
    out = _fin(aggp2, inv, h2, W_l2, b_l2.reshape(1, D), W_r2,
               W_cls, b_cls.reshape(1, N_CLS))
    return out[:n]


# speedup vs baseline: 11.4456x; 1.0211x over previous
import functools

import jax
import jax.numpy as jnp
from jax import lax
from jax.experimental import pallas as pl
from jax.experimental.pallas import tpu as pltpu
from jax.experimental.pallas import tpu_sc as plsc

N_NODES = 10000
N_PAD = 10240
D = 128
E = 320000
N_CLS = 40

NC, NS = 2, 16
NW = NC * NS
E_PER_TILE = E // NW
CH = 80
N_CHUNKS = E_PER_TILE // CH
N_PAIRS = (N_CHUNKS - 1) // 2
ROWS_PER_TILE = N_PAD // NS
BLK = 1024


def _sc_agg_body(with_deg, hl_hbm, src_hbm, dst_hbm, aggp_hbm, degp_hbm,
                 agg_sh, deg_sh, sall, dall, rows0, rows1, ones,
                 sem0, sem1, dsem):
    cid = lax.axis_index("c")
    sid = lax.axis_index("s")
    wid = cid * NS + sid

    pltpu.async_copy(src_hbm.at[wid], sall, sem0)
    pltpu.async_copy(dst_hbm.at[wid], dall, sem1)

    zero16 = jnp.zeros((16,), jnp.float32)
    for r in range(CH):
        for j in range(D // 16):
            rows1[r, pl.ds(j * 16, 16)] = zero16
    if with_deg:
        one16 = jnp.ones((16,), jnp.float32)
        for j in range(CH // 16):
            ones[pl.ds(j * 16, 16)] = one16
        ones[pl.ds(CH - 16, 16)] = one16

    def start(c, rows, sem):
        pltpu.async_copy(hl_hbm.at[sall.at[pl.ds(c * CH, CH)]], rows, sem)

    pltpu.make_async_copy(src_hbm.at[wid], sall, sem0).wait()
    start(0, rows0, sem0)

    r0 = sid * ROWS_PER_TILE
    for j in range(ROWS_PER_TILE // CH):
        pltpu.async_copy(rows1, agg_sh.at[pl.ds(r0 + j * CH, CH)], dsem)
    if with_deg:
        for j in range(ROWS_PER_TILE // D):
            pltpu.async_copy(rows1.at[0], deg_sh.at[pl.ds(r0 + j * D, D)], dsem)
    for j in range(ROWS_PER_TILE // CH):
        pltpu.make_async_copy(rows1, agg_sh.at[pl.ds(r0 + j * CH, CH)], dsem).wait()
    if with_deg:
        for j in range(ROWS_PER_TILE // D):
            pltpu.make_async_copy(rows1.at[0], deg_sh.at[pl.ds(r0 + j * D, D)], dsem).wait()
    pltpu.make_async_copy(dst_hbm.at[wid], dall, sem1).wait()
    plsc.subcore_barrier()

    def finish(c, rows, sem):
        pltpu.make_async_copy(hl_hbm.at[sall.at[pl.ds(c * CH, CH)]],
                              rows, sem).wait()
        pltpu.sync_copy(rows, agg_sh.at[dall.at[c]], add=True)
        if with_deg:
            pltpu.async_copy(ones, deg_sh.at[dall.at[c]], dsem, add=True)

    def pair_body(i, carry):
        c = 2 * i
        start(c + 1, rows1, sem1)
        finish(c, rows0, sem0)
        start(c + 2, rows0, sem0)
        finish(c + 1, rows1, sem1)
        return carry

    lax.fori_loop(0, N_PAIRS, pair_body, 0)
    finish(N_CHUNKS - 1, rows0, sem0)
    if with_deg:
        def drain_body(c, carry):
            pltpu.make_async_copy(ones, deg_sh.at[dall.at[c]], dsem).wait()
            return carry

        lax.fori_loop(0, N_CHUNKS, drain_body, 0)
    plsc.subcore_barrier()

    pltpu.async_copy(agg_sh.at[pl.ds(r0, ROWS_PER_TILE)],
                     aggp_hbm.at[cid, pl.ds(r0, ROWS_PER_TILE)], sem0)
    if with_deg:
        pltpu.async_copy(deg_sh.at[pl.ds(r0, ROWS_PER_TILE)],
                         degp_hbm.at[cid, pl.ds(r0, ROWS_PER_TILE)], sem1)
    pltpu.make_async_copy(agg_sh.at[pl.ds(r0, ROWS_PER_TILE)],
                          aggp_hbm.at[cid, pl.ds(r0, ROWS_PER_TILE)], sem0).wait()
    if with_deg:
        pltpu.make_async_copy(deg_sh.at[pl.ds(r0, ROWS_PER_TILE)],
                              degp_hbm.at[cid, pl.ds(r0, ROWS_PER_TILE)], sem1).wait()


@functools.lru_cache(maxsize=None)
def _sc_agg(with_deg):
    return pl.kernel(
        functools.partial(_sc_agg_body, with_deg),
        out_type=(jax.ShapeDtypeStruct((NC, N_PAD, D), jnp.float32),
                  jax.ShapeDtypeStruct((NC, N_PAD), jnp.float32)),
        mesh=plsc.VectorSubcoreMesh(core_axis_name="c", subcore_axis_name="s"),
        scratch_types=(
            pltpu.VMEM_SHARED((N_PAD, D), jnp.float32),
            pltpu.VMEM_SHARED((N_PAD,), jnp.float32),
            pltpu.VMEM((E_PER_TILE,), jnp.int32),
            pltpu.VMEM((N_CHUNKS, CH), jnp.int32),
            pltpu.VMEM((CH, D), jnp.float32),
            pltpu.VMEM((CH, D), jnp.float32),
            pltpu.VMEM((CH,), jnp.float32),
            pltpu.SemaphoreType.DMA,
            pltpu.SemaphoreType.DMA,
            pltpu.SemaphoreType.DMA,
        ),
    )


def _mid_body(aggp_ref, degp_ref, x_ref, wl_ref, bl_ref, wr_ref,
              h2_ref, inv_ref):
    p = aggp_ref[0] + aggp_ref[1]
    dsum = degp_ref[0] + degp_ref[1]
    inv = jnp.where(dsum > 0, 1.0 / jnp.maximum(dsum, 1.0), 0.0)
    mask = jnp.where(dsum > 0, 1.0, 0.0)
    agg = jnp.dot(p * inv, wl_ref[...], preferred_element_type=jnp.float32) + bl_ref[...] * mask
    hr = jnp.dot(x_ref[...], wr_ref[...], preferred_element_type=jnp.float32)
    h2_ref[...] = jnp.maximum(agg + hr, 0.0)
    inv_ref[...] = inv


def _mid(aggp, degp, xp, wl, bl, wr):
    return pl.pallas_call(
        _mid_body,
        grid=(N_PAD // BLK,),
        in_specs=[pl.BlockSpec((NC, BLK, D), lambda i: (0, i, 0)),
                  pl.BlockSpec((NC, BLK, 1), lambda i: (0, i, 0)),
                  pl.BlockSpec((BLK, D), lambda i: (i, 0)),
                  pl.BlockSpec((D, D), lambda i: (0, 0)),
                  pl.BlockSpec((1, D), lambda i: (0, 0)),
                  pl.BlockSpec((D, D), lambda i: (0, 0))],
        out_specs=[pl.BlockSpec((BLK, D), lambda i: (i, 0)),
                   pl.BlockSpec((BLK, 1), lambda i: (i, 0))],
        out_shape=[jax.ShapeDtypeStruct((N_PAD, D), jnp.float32),
                   jax.ShapeDtypeStruct((N_PAD, 1), jnp.float32)],
    )(aggp, degp, xp, wl, bl, wr)


def _fin_body(aggp_ref, inv_ref, h2_ref, wl_ref, bl_ref, wr_ref, wc_ref,
              bc_ref, out_ref):
    p = aggp_ref[0] + aggp_ref[1]
    inv = inv_ref[...]
    mask = jnp.where(inv > 0, 1.0, 0.0)
    agg = jnp.dot(p * inv, wl_ref[...], preferred_element_type=jnp.float32) + bl_ref[...] * mask
    hr = jnp.dot(h2_ref[...], wr_ref[...], preferred_element_type=jnp.float32)
    h3 = jnp.maximum(agg + hr, 0.0)
    out_ref[...] = jnp.dot(h3, wc_ref[...], preferred_element_type=jnp.float32) + bc_ref[...]


def _fin(aggp, inv, h2, wl, bl, wr, wc, bc):
    return pl.pallas_call(
        _fin_body,
        grid=(N_PAD // BLK,),
        in_specs=[pl.BlockSpec((NC, BLK, D), lambda i: (0, i, 0)),
                  pl.BlockSpec((BLK, 1), lambda i: (i, 0)),
                  pl.BlockSpec((BLK, D), lambda i: (i, 0)),
                  pl.BlockSpec((D, D), lambda i: (0, 0)),
                  pl.BlockSpec((1, D), lambda i: (0, 0)),
                  pl.BlockSpec((D, D), lambda i: (0, 0)),
                  pl.BlockSpec((D, N_CLS), lambda i: (0, 0)),
                  pl.BlockSpec((1, N_CLS), lambda i: (0, 0))],
        out_specs=pl.BlockSpec((BLK, N_CLS), lambda i: (i, 0)),
        out_shape=jax.ShapeDtypeStruct((N_PAD, N_CLS), jnp.float32),
    )(aggp, inv, h2, wl, bl, wr, wc, bc)


def kernel(x, edge_index, W_l1, b_l1, W_r1, W_l2, b_l2, W_r2, W_cls, b_cls):
    n = x.shape[0]
    src = edge_index[0].astype(jnp.int32).reshape(NW, E_PER_TILE)
    dst = edge_index[1].astype(jnp.int32).reshape(NW, N_CHUNKS, CH)
    xp = jnp.zeros((N_PAD, D), jnp.float32).at[:n].set(x)

    aggp1, degp1 = _sc_agg(True)(xp, src, dst)
    degp3 = degp1.reshape(NC, N_PAD, 1)
    h2, inv = _mid(aggp1, degp3, xp, W_l1, b_l1.reshape(1, D), W_r1)
    aggp2, _ = _sc_agg(False)(h2, src, dst)
    out = _fin(aggp2, inv, h2, W_l2, b_l2.reshape(1, D), W_r2,
               W_cls, b_cls.reshape(1, N_CLS))
    return out[:n]
